# baseline (speedup 1.0000x reference)
import functools

import jax
import jax.numpy as jnp
import numpy as np
from jax import lax
from jax.experimental import pallas as pl
from jax.experimental.pallas import tpu as pltpu
from jax.experimental.pallas import tpu_sc as plsc

_EPS = float(np.finfo(np.float64).eps)
_TOPK = 10
_L = 16


def _embed_affinity_body(sup_ref, qry_ref, wq_ref, wv_ref, e_ref, ev_ref,
                         eqacc):
    i = pl.program_id(0)
    blk = sup_ref.shape[0]
    n = eqacc.shape[0]
    x = lax.cond(i == 0, lambda: sup_ref[...], lambda: qry_ref[...])
    xb = x.astype(jnp.bfloat16).astype(jnp.float32)
    m = jnp.mean(xb, axis=-1)
    wqb = wq_ref[...].astype(jnp.bfloat16).astype(jnp.float32)
    wvb = wv_ref[...].astype(jnp.bfloat16).astype(jnp.float32)
    dn = (((1,), (1,)), ((), ()))
    eqb = lax.dot_general(m, wqb, dn, preferred_element_type=jnp.float32,
                          precision=lax.Precision.HIGHEST)
    ev_ref[...] = lax.dot_general(
        m, wvb, dn, preferred_element_type=jnp.float32,
        precision=lax.Precision.HIGHEST)[None]
    off = i * blk
    a = pl.multiple_of(off - 4 * (i % 2), 8)
    old = eqacc[pl.ds(a, blk + 4), :]
    merged = lax.cond(
        i % 2 == 0,
        lambda: jnp.concatenate([eqb, old[blk:blk + 4]], axis=0),
        lambda: jnp.concatenate([old[0:4], eqb], axis=0))
    eqacc[pl.ds(a, blk + 4), :] = merged

    @pl.when(i == pl.num_programs(0) - 1)
    def _():
        eq = eqacc[...]
        g = lax.dot_general(eq, eq, (((1,), (1,)), ((), ())),
                            preferred_element_type=jnp.float32,
                            precision=lax.Precision.HIGHEST)
        r = jnp.sum(eq * eq, axis=1)
        w = r[:, None] + r[None, :] - 2.0 * g
        ii = lax.broadcasted_iota(jnp.int32, (n, n), 0)
        jj = lax.broadcasted_iota(jnp.int32, (n, n), 1)
        w = jnp.where(ii == jj, 0.0, jnp.maximum(w, 0.0))
        nz = w != 0.0
        cnt = jnp.sum(nz.astype(jnp.float32))
        mean = jnp.sum(w) / cnt
        var = jnp.sum(jnp.where(nz, (w - mean) ** 2, 0.0)) / (cnt - 1.0)
        std = jnp.sqrt(var)
        e = jnp.exp(-(w / std) / 2.0)
        pad = e_ref.shape[1] - n
        e_ref[...] = jnp.concatenate(
            [e, jnp.zeros((n, pad), jnp.float32)], axis=1)


def _embed_affinity(sup, qry, wq, wv, npad):
    ns, c, hw = sup.shape
    nq = qry.shape[0]
    blk = ns
    g = 1 + nq // blk
    n = ns + nq
    e, ev = pl.pallas_call(
        _embed_affinity_body,
        grid=(g,),
        in_specs=[
            pl.BlockSpec((blk, c, hw), lambda i: (0, 0, 0)),
            pl.BlockSpec((blk, c, hw), lambda i: (jnp.maximum(i - 1, 0), 0, 0)),
            pl.BlockSpec((c, c), lambda i: (0, 0)),
            pl.BlockSpec((c, c), lambda i: (0, 0)),
        ],
        out_specs=[
            pl.BlockSpec((n, npad), lambda i: (0, 0)),
            pl.BlockSpec((1, blk, c), lambda i: (i, 0, 0)),
        ],
        out_shape=[
            jax.ShapeDtypeStruct((n, npad), jnp.float32),
            jax.ShapeDtypeStruct((g, blk, c), jnp.float32),
        ],
        scratch_shapes=[pltpu.VMEM((n, c), jnp.float32)],
    )(sup, qry, wq, wv)
    return e, ev.reshape(n, c)


def _embed_body(x_ref, aq_ref, av_ref, eq_ref, ev_ref):
    xb = x_ref[0].astype(jnp.bfloat16)
    dn = (((1,), (0,)), ((), ()))
    eq = lax.dot_general(xb, aq_ref[...], dn,
                         preferred_element_type=jnp.float32)
    ev = lax.dot_general(xb, av_ref[...], dn,
                         preferred_element_type=jnp.float32)
    eq_ref[...] = (eq * jnp.float32(1.0 / 49.0))[None]
    ev_ref[...] = (ev * jnp.float32(1.0 / 49.0))[None]


def _embed(x, aq, av, block):
    n, chw = x.shape
    c = aq.shape[1]
    g = n // block
    x = x.reshape(g, block, chw)
    eq, ev = pl.pallas_call(
        _embed_body,
        grid=(g,),
        in_specs=[
            pl.BlockSpec((1, block, chw), lambda i: (i, 0, 0)),
            pl.BlockSpec((chw, c), lambda i: (0, 0)),
            pl.BlockSpec((chw, c), lambda i: (0, 0)),
        ],
        out_specs=[
            pl.BlockSpec((1, block, c), lambda i: (i, 0, 0)),
            pl.BlockSpec((1, block, c), lambda i: (i, 0, 0)),
        ],
        out_shape=[
            jax.ShapeDtypeStruct((g, block, c), jnp.float32),
            jax.ShapeDtypeStruct((g, block, c), jnp.float32),
        ],
    )(x, aq, av)
    return eq.reshape(n, c), ev.reshape(n, c)


def _affinity_body(eq_ref, e_ref):
    eq = eq_ref[...]
    n = eq.shape[0]
    g = lax.dot_general(eq, eq, (((1,), (1,)), ((), ())),
                        preferred_element_type=jnp.float32,
                                  precision=lax.Precision.HIGHEST)
    r = jnp.sum(eq * eq, axis=1)
    w = r[:, None] + r[None, :] - 2.0 * g
    ii = lax.broadcasted_iota(jnp.int32, (n, n), 0)
    jj = lax.broadcasted_iota(jnp.int32, (n, n), 1)
    w = jnp.where(ii == jj, 0.0, jnp.maximum(w, 0.0))
    nz = w != 0.0
    cnt = jnp.sum(nz.astype(jnp.float32))
    mean = jnp.sum(w) / cnt
    var = jnp.sum(jnp.where(nz, (w - mean) ** 2, 0.0)) / (cnt - 1.0)
    std = jnp.sqrt(var)
    e = jnp.exp(-(w / std) / 2.0)
    pad = e_ref.shape[1] - n
    e_ref[...] = jnp.concatenate([e, jnp.zeros((n, pad), jnp.float32)], axis=1)


def _affinity(eq, npad):
    n = eq.shape[0]
    return pl.pallas_call(
        _affinity_body,
        out_shape=jax.ShapeDtypeStruct((n, npad), jnp.float32),
    )(eq)


def _topk(e):
    n, npad = e.shape
    nc, ns = 2, 16
    nw = nc * ns
    rpw = (n + nw - 1) // nw
    nchunks = npad // _L
    mesh = plsc.VectorSubcoreMesh(core_axis_name="c", subcore_axis_name="s",
                                  num_cores=nc, num_subcores=ns)

    @functools.partial(
        pl.kernel,
        mesh=mesh,
        out_type=jax.ShapeDtypeStruct((n, _L), jnp.int32),
        scratch_types=[
            pltpu.VMEM((npad,), jnp.float32),
            pltpu.VMEM((_L,), jnp.int32),
        ],
        compiler_params=pltpu.CompilerParams(needs_layout_passes=False),
    )
    def k(e_hbm, out_hbm, row_v, idx_v):
        wid = lax.axis_index("s") * nc + lax.axis_index("c")
        lane = lax.iota(jnp.int32, _L)

        def body(r, carry):
            row = wid * rpw + r

            @pl.when(row < n)
            def _():
                pltpu.sync_copy(e_hbm.at[row], row_v)
                rk = row_v[pl.ds(0, _L)]
                rv = lane
                rk, rv = plsc.sort_key_val(rk, rv)
                for c in range(1, nchunks):
                    ck = row_v[pl.ds(c * _L, _L)]
                    cv = lane + c * _L
                    ck, cv = plsc.sort_key_val(ck, cv, descending=True)
                    take = rk >= ck
                    nk = jnp.maximum(rk, ck)
                    nv = jnp.where(take, rv, cv)
                    rk, rv = plsc.sort_key_val(nk, nv)
                idx_v[...] = rv
                pltpu.sync_copy(idx_v, out_hbm.at[row])

            return carry

        lax.fori_loop(0, rpw, body, 0)

    return k(e)


def _propagate_body(e_ref, idx_ref, ev_ref, out_ref):
    n = idx_ref.shape[0]
    e = e_ref[...][:, :n]
    idx = idx_ref[...]
    jj = lax.broadcasted_iota(jnp.int32, (n, n), 1)
    m = jnp.zeros((n, n), jnp.bool_)
    for k in range(_L - _TOPK, _L):
        m = m | (idx[:, k:k + 1] == jj)
    wm0 = jnp.where(m, e, 0.0)
    wm = jnp.maximum(wm0, wm0.T)
    d = jnp.sum(wm, axis=0)
    dsi = jnp.sqrt(1.0 / (d + _EPS))
    s = dsi[:, None] * wm * dsi[None, :]
    out_ref[...] = lax.dot_general(s, ev_ref[...], (((1,), (0,)), ((), ())),
                                   preferred_element_type=jnp.float32,
                                  precision=lax.Precision.HIGHEST)


def _propagate(e, idx, ev):
    n, c = ev.shape
    return pl.pallas_call(
        _propagate_body,
        out_shape=jax.ShapeDtypeStruct((n, c), jnp.float32),
    )(e, idx, ev)


def kernel(support, s_labels, query, q_labels, Wq, Wv):
    ns, c = support.shape[0], support.shape[1]
    nq = query.shape[0]
    hw = support.shape[2] * support.shape[3]
    n = ns + nq
    npad = ((n + _L - 1) // _L) * _L
    if npad == n:
        npad += _L
    aq = jnp.repeat(Wq.astype(jnp.bfloat16).T, hw, axis=0)
    av = jnp.repeat(Wv.astype(jnp.bfloat16).T, hw, axis=0)
    eq_s, ev_s = _embed(support.reshape(ns, c * hw), aq, av, ns)
    eq_q, ev_q = _embed(query.reshape(nq, c * hw), aq, av, nq // 5)
    emb_q = jnp.concatenate([eq_s, eq_q], axis=0)
    emb_v = jnp.concatenate([ev_s, ev_q], axis=0)
    e = _affinity(emb_q, npad)
    idx = _topk(e)
    return _propagate(e, idx, emb_v)


# speedup vs baseline: 1.4769x; 1.4769x over previous
import functools

import jax
import jax.numpy as jnp
import numpy as np
from jax import lax
from jax.experimental import pallas as pl
from jax.experimental.pallas import tpu as pltpu
from jax.experimental.pallas import tpu_sc as plsc

_EPS = float(np.finfo(np.float64).eps)
_TOPK = 10
_L = 16


def _embed_affinity_body(sup_ref, qry_ref, wq_ref, wv_ref, e_ref, ev_ref,
                         eqacc):
    i = pl.program_id(0)
    blk = sup_ref.shape[0]
    n = eqacc.shape[0]
    x = lax.cond(i == 0, lambda: sup_ref[...], lambda: qry_ref[...])
    xb = x.astype(jnp.bfloat16).astype(jnp.float32)
    m = jnp.mean(xb, axis=-1)
    wqb = wq_ref[...].astype(jnp.bfloat16).astype(jnp.float32)
    wvb = wv_ref[...].astype(jnp.bfloat16).astype(jnp.float32)
    dn = (((1,), (1,)), ((), ()))
    eqb = lax.dot_general(m, wqb, dn, preferred_element_type=jnp.float32,
                          precision=lax.Precision.HIGHEST)
    ev_ref[...] = lax.dot_general(
        m, wvb, dn, preferred_element_type=jnp.float32,
        precision=lax.Precision.HIGHEST)[None]
    off = i * blk
    a = pl.multiple_of(off - 4 * (i % 2), 8)
    old = eqacc[pl.ds(a, blk + 4), :]
    merged = lax.cond(
        i % 2 == 0,
        lambda: jnp.concatenate([eqb, old[blk:blk + 4]], axis=0),
        lambda: jnp.concatenate([old[0:4], eqb], axis=0))
    eqacc[pl.ds(a, blk + 4), :] = merged

    @pl.when(i == pl.num_programs(0) - 1)
    def _():
        eq = eqacc[...]
        g = lax.dot_general(eq, eq, (((1,), (1,)), ((), ())),
                            preferred_element_type=jnp.float32,
                            precision=lax.Precision.HIGHEST)
        r = jnp.sum(eq * eq, axis=1)
        w = r[:, None] + r[None, :] - 2.0 * g
        ii = lax.broadcasted_iota(jnp.int32, (n, n), 0)
        jj = lax.broadcasted_iota(jnp.int32, (n, n), 1)
        w = jnp.where(ii == jj, 0.0, jnp.maximum(w, 0.0))
        nz = w != 0.0
        cnt = jnp.sum(nz.astype(jnp.float32))
        mean = jnp.sum(w) / cnt
        var = jnp.sum(jnp.where(nz, (w - mean) ** 2, 0.0)) / (cnt - 1.0)
        std = jnp.sqrt(var)
        e = jnp.exp(-(w / std) / 2.0)
        pad = e_ref.shape[1] - n
        e_ref[...] = jnp.concatenate(
            [e, jnp.zeros((n, pad), jnp.float32)], axis=1)


def _embed_affinity(sup, qry, wq, wv, npad):
    ns, c, hw = sup.shape
    nq = qry.shape[0]
    blk = ns
    g = 1 + nq // blk
    n = ns + nq
    e, ev = pl.pallas_call(
        _embed_affinity_body,
        grid=(g,),
        in_specs=[
            pl.BlockSpec((blk, c, hw), lambda i: (0, 0, 0)),
            pl.BlockSpec((blk, c, hw), lambda i: (jnp.maximum(i - 1, 0), 0, 0)),
            pl.BlockSpec((c, c), lambda i: (0, 0)),
            pl.BlockSpec((c, c), lambda i: (0, 0)),
        ],
        out_specs=[
            pl.BlockSpec((n, npad), lambda i: (0, 0)),
            pl.BlockSpec((1, blk, c), lambda i: (i, 0, 0)),
        ],
        out_shape=[
            jax.ShapeDtypeStruct((n, npad), jnp.float32),
            jax.ShapeDtypeStruct((g, blk, c), jnp.float32),
        ],
        scratch_shapes=[pltpu.VMEM((n, c), jnp.float32)],
    )(sup, qry, wq, wv)
    return e, ev.reshape(n, c)


def _embed_body(x_ref, wq_ref, wv_ref, eq_ref, ev_ref):
    xb = x_ref[...].astype(jnp.bfloat16).astype(jnp.float32)
    m = jnp.mean(xb, axis=-1)
    dn = (((1,), (1,)), ((), ()))
    wqb = wq_ref[...].astype(jnp.bfloat16).astype(jnp.float32)
    wvb = wv_ref[...].astype(jnp.bfloat16).astype(jnp.float32)
    eq_ref[...] = lax.dot_general(m, wqb, dn,
                                  preferred_element_type=jnp.float32,
                                  precision=lax.Precision.HIGHEST)[None]
    ev_ref[...] = lax.dot_general(m, wvb, dn,
                                  preferred_element_type=jnp.float32,
                                  precision=lax.Precision.HIGHEST)[None]


def _embed(x, wq, wv, block):
    n, c, hw = x.shape
    g = n // block
    eq, ev = pl.pallas_call(
        _embed_body,
        grid=(g,),
        in_specs=[
            pl.BlockSpec((block, c, hw), lambda i: (i, 0, 0)),
            pl.BlockSpec((c, c), lambda i: (0, 0)),
            pl.BlockSpec((c, c), lambda i: (0, 0)),
        ],
        out_specs=[
            pl.BlockSpec((1, block, c), lambda i: (i, 0, 0)),
            pl.BlockSpec((1, block, c), lambda i: (i, 0, 0)),
        ],
        out_shape=[
            jax.ShapeDtypeStruct((g, block, c), jnp.float32),
            jax.ShapeDtypeStruct((g, block, c), jnp.float32),
        ],
    )(x, wq, wv)
    return eq.reshape(n, c), ev.reshape(n, c)


def _affinity_body(eq_ref, e_ref):
    eq = eq_ref[...]
    n = eq.shape[0]
    g = lax.dot_general(eq, eq, (((1,), (1,)), ((), ())),
                        preferred_element_type=jnp.float32,
                                  precision=lax.Precision.HIGHEST)
    r = jnp.sum(eq * eq, axis=1)
    w = r[:, None] + r[None, :] - 2.0 * g
    ii = lax.broadcasted_iota(jnp.int32, (n, n), 0)
    jj = lax.broadcasted_iota(jnp.int32, (n, n), 1)
    w = jnp.where(ii == jj, 0.0, jnp.maximum(w, 0.0))
    nz = w != 0.0
    cnt = jnp.sum(nz.astype(jnp.float32))
    mean = jnp.sum(w) / cnt
    var = jnp.sum(jnp.where(nz, (w - mean) ** 2, 0.0)) / (cnt - 1.0)
    std = jnp.sqrt(var)
    e = jnp.exp(-(w / std) / 2.0)
    pad = e_ref.shape[1] - n
    e_ref[...] = jnp.concatenate([e, jnp.zeros((n, pad), jnp.float32)], axis=1)


def _affinity(eq, npad):
    n = eq.shape[0]
    return pl.pallas_call(
        _affinity_body,
        out_shape=jax.ShapeDtypeStruct((n, npad), jnp.float32),
    )(eq)


def _topk(e):
    n, npad = e.shape
    nc, ns = 2, 16
    nw = nc * ns
    rpw = (n + nw - 1) // nw
    nchunks = npad // _L
    mesh = plsc.VectorSubcoreMesh(core_axis_name="c", subcore_axis_name="s",
                                  num_cores=nc, num_subcores=ns)

    @functools.partial(
        pl.kernel,
        mesh=mesh,
        out_type=jax.ShapeDtypeStruct((nw, rpw, _L), jnp.int32),
        scratch_types=[
            pltpu.VMEM((npad,), jnp.float32),
            pltpu.VMEM((npad,), jnp.float32),
            pltpu.VMEM((rpw, _L), jnp.int32),
            pltpu.SemaphoreType.DMA,
            pltpu.SemaphoreType.DMA,
        ],
        compiler_params=pltpu.CompilerParams(needs_layout_passes=False),
    )
    def k(e_hbm, out_hbm, row0, row1, idxb, sem0, sem1):
        wid = lax.axis_index("s") * nc + lax.axis_index("c")
        base = wid * rpw
        nrow = jnp.minimum(rpw, n - base)
        lane = lax.iota(jnp.int32, _L)

        def merge_row(buf):
            rk = buf[pl.ds(0, _L)]
            rv = lane
            rk, rv = plsc.sort_key_val(rk, rv)
            for c in range(1, nchunks):
                ck = buf[pl.ds(c * _L, _L)]
                cv = lane + c * _L
                ck, cv = plsc.sort_key_val(ck, cv, descending=True)
                take = rk >= ck
                nk = jnp.maximum(rk, ck)
                nv = jnp.where(take, rv, cv)
                rk, rv = plsc.sort_key_val(nk, nv)
            return rv

        @pl.when(nrow > 0)
        def _():
            pltpu.make_async_copy(e_hbm.at[base], row0, sem0).start()

        @pl.when(nrow > 1)
        def _():
            pltpu.make_async_copy(e_hbm.at[base + 1], row1, sem1).start()

        def body(it, carry):
            r0 = it * 2
            r1 = r0 + 1

            @pl.when(r0 < nrow)
            def _():
                pltpu.make_async_copy(e_hbm.at[base + r0], row0, sem0).wait()
                idxb[r0, :] = merge_row(row0)

                @pl.when(r0 + 2 < nrow)
                def _():
                    pltpu.make_async_copy(
                        e_hbm.at[base + r0 + 2], row0, sem0).start()

            @pl.when(r1 < nrow)
            def _():
                pltpu.make_async_copy(e_hbm.at[base + r1], row1, sem1).wait()
                idxb[r1, :] = merge_row(row1)

                @pl.when(r1 + 2 < nrow)
                def _():
                    pltpu.make_async_copy(
                        e_hbm.at[base + r1 + 2], row1, sem1).start()

            return carry

        lax.fori_loop(0, (rpw + 1) // 2, body, 0)
        pltpu.sync_copy(idxb, out_hbm.at[wid])

    return k(e).reshape(nw * rpw, _L)[:n]


def _propagate_body(e_ref, idx_ref, ev_ref, out_ref):
    n = idx_ref.shape[0]
    e = e_ref[...][:, :n]
    idx = idx_ref[...]
    jj = lax.broadcasted_iota(jnp.int32, (n, n), 1)
    m = jnp.zeros((n, n), jnp.bool_)
    for k in range(_L - _TOPK, _L):
        m = m | (idx[:, k:k + 1] == jj)
    wm0 = jnp.where(m, e, 0.0)
    wm = jnp.maximum(wm0, wm0.T)
    d = jnp.sum(wm, axis=0)
    dsi = jnp.sqrt(1.0 / (d + _EPS))
    s = dsi[:, None] * wm * dsi[None, :]
    out_ref[...] = lax.dot_general(s, ev_ref[...], (((1,), (0,)), ((), ())),
                                   preferred_element_type=jnp.float32,
                                  precision=lax.Precision.HIGHEST)


def _propagate(e, idx, ev):
    n, c = ev.shape
    return pl.pallas_call(
        _propagate_body,
        out_shape=jax.ShapeDtypeStruct((n, c), jnp.float32),
    )(e, idx, ev)


def kernel(support, s_labels, query, q_labels, Wq, Wv):
    ns, c = support.shape[0], support.shape[1]
    nq = query.shape[0]
    hw = support.shape[2] * support.shape[3]
    n = ns + nq
    npad = ((n + _L - 1) // _L) * _L
    if npad == n:
        npad += _L
    eq_s, ev_s = _embed(support.reshape(ns, c, hw), Wq, Wv, ns)
    eq_q, ev_q = _embed(query.reshape(nq, c, hw), Wq, Wv, nq // 5)
    emb_q = jnp.concatenate([eq_s, eq_q], axis=0)
    emb_v = jnp.concatenate([ev_s, ev_q], axis=0)
    e = _affinity(emb_q, npad)
    idx = _topk(e)
    return _propagate(e, idx, emb_v)


# speedup vs baseline: 1.4906x; 1.0093x over previous
import functools

import jax
import jax.numpy as jnp
import numpy as np
from jax import lax
from jax.experimental import pallas as pl
from jax.experimental.pallas import tpu as pltpu
from jax.experimental.pallas import tpu_sc as plsc

_EPS = float(np.finfo(np.float64).eps)
_TOPK = 10
_L = 16


def _embed_affinity_body(sup_ref, qry_ref, wq_ref, wv_ref, e_ref, ev_ref,
                         eqacc):
    i = pl.program_id(0)
    blk = sup_ref.shape[0]
    n = eqacc.shape[0]
    x = lax.cond(i == 0, lambda: sup_ref[...], lambda: qry_ref[...])
    xb = x.astype(jnp.bfloat16).astype(jnp.float32)
    m = jnp.mean(xb, axis=-1)
    wqb = wq_ref[...].astype(jnp.bfloat16).astype(jnp.float32)
    wvb = wv_ref[...].astype(jnp.bfloat16).astype(jnp.float32)
    dn = (((1,), (1,)), ((), ()))
    eqb = lax.dot_general(m, wqb, dn, preferred_element_type=jnp.float32,
                          precision=lax.Precision.HIGHEST)
    ev_ref[...] = lax.dot_general(
        m, wvb, dn, preferred_element_type=jnp.float32,
        precision=lax.Precision.HIGHEST)[None]
    off = i * blk
    a = pl.multiple_of(off - 4 * (i % 2), 8)
    old = eqacc[pl.ds(a, blk + 4), :]
    merged = lax.cond(
        i % 2 == 0,
        lambda: jnp.concatenate([eqb, old[blk:blk + 4]], axis=0),
        lambda: jnp.concatenate([old[0:4], eqb], axis=0))
    eqacc[pl.ds(a, blk + 4), :] = merged

    @pl.when(i == pl.num_programs(0) - 1)
    def _():
        eq = eqacc[...]
        g = lax.dot_general(eq, eq, (((1,), (1,)), ((), ())),
                            preferred_element_type=jnp.float32,
                            precision=lax.Precision.HIGHEST)
        r = jnp.sum(eq * eq, axis=1)
        w = r[:, None] + r[None, :] - 2.0 * g
        ii = lax.broadcasted_iota(jnp.int32, (n, n), 0)
        jj = lax.broadcasted_iota(jnp.int32, (n, n), 1)
        w = jnp.where(ii == jj, 0.0, jnp.maximum(w, 0.0))
        nz = w != 0.0
        cnt = jnp.sum(nz.astype(jnp.float32))
        mean = jnp.sum(w) / cnt
        var = jnp.sum(jnp.where(nz, (w - mean) ** 2, 0.0)) / (cnt - 1.0)
        std = jnp.sqrt(var)
        e = jnp.exp(-(w / std) / 2.0)
        pad = e_ref.shape[1] - n
        e_ref[...] = jnp.concatenate(
            [e, jnp.zeros((n, pad), jnp.float32)], axis=1)


def _embed_affinity(sup, qry, wq, wv, npad):
    ns, c, hw = sup.shape
    nq = qry.shape[0]
    blk = ns
    g = 1 + nq // blk
    n = ns + nq
    e, ev = pl.pallas_call(
        _embed_affinity_body,
        grid=(g,),
        in_specs=[
            pl.BlockSpec((blk, c, hw), lambda i: (0, 0, 0)),
            pl.BlockSpec((blk, c, hw), lambda i: (jnp.maximum(i - 1, 0), 0, 0)),
            pl.BlockSpec((c, c), lambda i: (0, 0)),
            pl.BlockSpec((c, c), lambda i: (0, 0)),
        ],
        out_specs=[
            pl.BlockSpec((n, npad), lambda i: (0, 0)),
            pl.BlockSpec((1, blk, c), lambda i: (i, 0, 0)),
        ],
        out_shape=[
            jax.ShapeDtypeStruct((n, npad), jnp.float32),
            jax.ShapeDtypeStruct((g, blk, c), jnp.float32),
        ],
        scratch_shapes=[pltpu.VMEM((n, c), jnp.float32)],
    )(sup, qry, wq, wv)
    return e, ev.reshape(n, c)


def _embed_body(x_ref, wq_ref, wv_ref, eq_ref, ev_ref):
    xb = x_ref[...].astype(jnp.bfloat16).astype(jnp.float32)
    m = jnp.mean(xb, axis=-1)
    dn = (((1,), (1,)), ((), ()))
    wqb = wq_ref[...].astype(jnp.bfloat16).astype(jnp.float32)
    wvb = wv_ref[...].astype(jnp.bfloat16).astype(jnp.float32)
    eq_ref[...] = lax.dot_general(m, wqb, dn,
                                  preferred_element_type=jnp.float32,
                                  precision=lax.Precision.HIGHEST)[None]
    ev_ref[...] = lax.dot_general(m, wvb, dn,
                                  preferred_element_type=jnp.float32,
                                  precision=lax.Precision.HIGHEST)[None]


def _embed(x, wq, wv, block):
    n, c, hw = x.shape
    g = n // block
    eq, ev = pl.pallas_call(
        _embed_body,
        grid=(g,),
        in_specs=[
            pl.BlockSpec((block, c, hw), lambda i: (i, 0, 0)),
            pl.BlockSpec((c, c), lambda i: (0, 0)),
            pl.BlockSpec((c, c), lambda i: (0, 0)),
        ],
        out_specs=[
            pl.BlockSpec((1, block, c), lambda i: (i, 0, 0)),
            pl.BlockSpec((1, block, c), lambda i: (i, 0, 0)),
        ],
        out_shape=[
            jax.ShapeDtypeStruct((g, block, c), jnp.float32),
            jax.ShapeDtypeStruct((g, block, c), jnp.float32),
        ],
    )(x, wq, wv)
    return eq.reshape(n, c), ev.reshape(n, c)


def _affinity_body(eqs_ref, eqq_ref, e_ref):
    eq = jnp.concatenate([eqs_ref[...], eqq_ref[...]], axis=0)
    n = eq.shape[0]
    g = lax.dot_general(eq, eq, (((1,), (1,)), ((), ())),
                        preferred_element_type=jnp.float32,
                                  precision=lax.Precision.HIGHEST)
    r = jnp.sum(eq * eq, axis=1)
    w = r[:, None] + r[None, :] - 2.0 * g
    ii = lax.broadcasted_iota(jnp.int32, (n, n), 0)
    jj = lax.broadcasted_iota(jnp.int32, (n, n), 1)
    w = jnp.where(ii == jj, 0.0, jnp.maximum(w, 0.0))
    nz = w != 0.0
    cnt = jnp.sum(nz.astype(jnp.float32))
    mean = jnp.sum(w) / cnt
    var = jnp.sum(jnp.where(nz, (w - mean) ** 2, 0.0)) / (cnt - 1.0)
    std = jnp.sqrt(var)
    e = jnp.exp(-(w / std) / 2.0)
    pad = e_ref.shape[1] - n
    e_ref[...] = jnp.concatenate([e, jnp.zeros((n, pad), jnp.float32)], axis=1)


def _affinity(eq_s, eq_q, npad):
    n = eq_s.shape[0] + eq_q.shape[0]
    return pl.pallas_call(
        _affinity_body,
        out_shape=jax.ShapeDtypeStruct((n, npad), jnp.float32),
    )(eq_s, eq_q)


def _topk(e):
    n, npad = e.shape
    nc, ns = 2, 16
    nw = nc * ns
    rpw = (n + nw - 1) // nw
    nchunks = npad // _L
    mesh = plsc.VectorSubcoreMesh(core_axis_name="c", subcore_axis_name="s",
                                  num_cores=nc, num_subcores=ns)

    @functools.partial(
        pl.kernel,
        mesh=mesh,
        out_type=jax.ShapeDtypeStruct((nw, rpw, _L), jnp.int32),
        scratch_types=[
            pltpu.VMEM((npad,), jnp.float32),
            pltpu.VMEM((npad,), jnp.float32),
            pltpu.VMEM((rpw, _L), jnp.int32),
            pltpu.SemaphoreType.DMA,
            pltpu.SemaphoreType.DMA,
        ],
        compiler_params=pltpu.CompilerParams(needs_layout_passes=False),
    )
    def k(e_hbm, out_hbm, row0, row1, idxb, sem0, sem1):
        wid = lax.axis_index("s") * nc + lax.axis_index("c")
        base = wid * rpw
        nrow = jnp.minimum(rpw, n - base)
        lane = lax.iota(jnp.int32, _L)

        def merge_row(buf):
            rk = buf[pl.ds(0, _L)]
            rv = lane
            rk, rv = plsc.sort_key_val(rk, rv)
            for c in range(1, nchunks):
                ck = buf[pl.ds(c * _L, _L)]
                cv = lane + c * _L
                ck, cv = plsc.sort_key_val(ck, cv, descending=True)
                take = rk >= ck
                nk = jnp.maximum(rk, ck)
                nv = jnp.where(take, rv, cv)
                rk, rv = plsc.sort_key_val(nk, nv)
            return rv

        @pl.when(nrow > 0)
        def _():
            pltpu.make_async_copy(e_hbm.at[base], row0, sem0).start()

        @pl.when(nrow > 1)
        def _():
            pltpu.make_async_copy(e_hbm.at[base + 1], row1, sem1).start()

        def body(it, carry):
            r0 = it * 2
            r1 = r0 + 1

            @pl.when(r0 < nrow)
            def _():
                pltpu.make_async_copy(e_hbm.at[base + r0], row0, sem0).wait()
                idxb[r0, :] = merge_row(row0)

                @pl.when(r0 + 2 < nrow)
                def _():
                    pltpu.make_async_copy(
                        e_hbm.at[base + r0 + 2], row0, sem0).start()

            @pl.when(r1 < nrow)
            def _():
                pltpu.make_async_copy(e_hbm.at[base + r1], row1, sem1).wait()
                idxb[r1, :] = merge_row(row1)

                @pl.when(r1 + 2 < nrow)
                def _():
                    pltpu.make_async_copy(
                        e_hbm.at[base + r1 + 2], row1, sem1).start()

            return carry

        lax.fori_loop(0, (rpw + 1) // 2, body, 0)
        pltpu.sync_copy(idxb, out_hbm.at[wid])

    return k(e).reshape(nw * rpw, _L)[:n]


def _propagate_body(e_ref, idx_ref, evs_ref, evq_ref, out_ref):
    n = idx_ref.shape[0]
    e = e_ref[...][:, :n]
    ev = jnp.concatenate([evs_ref[...], evq_ref[...]], axis=0)
    idx = idx_ref[...]
    jj = lax.broadcasted_iota(jnp.int32, (n, n), 1)
    m = jnp.zeros((n, n), jnp.bool_)
    for k in range(_L - _TOPK, _L):
        m = m | (idx[:, k:k + 1] == jj)
    wm0 = jnp.where(m, e, 0.0)
    wm = jnp.maximum(wm0, wm0.T)
    d = jnp.sum(wm, axis=0)
    dsi = jnp.sqrt(1.0 / (d + _EPS))
    s = dsi[:, None] * wm * dsi[None, :]
    out_ref[...] = lax.dot_general(s, ev, (((1,), (0,)), ((), ())),
                                   preferred_element_type=jnp.float32,
                                  precision=lax.Precision.HIGHEST)


def _propagate(e, idx, ev_s, ev_q):
    n = ev_s.shape[0] + ev_q.shape[0]
    c = ev_s.shape[1]
    return pl.pallas_call(
        _propagate_body,
        out_shape=jax.ShapeDtypeStruct((n, c), jnp.float32),
    )(e, idx, ev_s, ev_q)


def kernel(support, s_labels, query, q_labels, Wq, Wv):
    ns, c = support.shape[0], support.shape[1]
    nq = query.shape[0]
    hw = support.shape[2] * support.shape[3]
    n = ns + nq
    npad = ((n + _L - 1) // _L) * _L
    if npad == n:
        npad += _L
    eq_s, ev_s = _embed(support.reshape(ns, c, hw), Wq, Wv, ns)
    eq_q, ev_q = _embed(query.reshape(nq, c, hw), Wq, Wv, nq // 5)
    e = _affinity(eq_s, eq_q, npad)
    idx = _topk(e)
    return _propagate(e, idx, ev_s, ev_q)


# speedup vs baseline: 1.4980x; 1.0049x over previous
import functools

import jax
import jax.numpy as jnp
import numpy as np
from jax import lax
from jax.experimental import pallas as pl
from jax.experimental.pallas import tpu as pltpu
from jax.experimental.pallas import tpu_sc as plsc

_EPS = float(np.finfo(np.float64).eps)
_TOPK = 10
_L = 16


def _embed_body(x_ref, wq_ref, wv_ref, eq_ref, ev_ref):
    xb = x_ref[...].astype(jnp.bfloat16).astype(jnp.float32)
    m = jnp.mean(xb, axis=-1)
    dn = (((1,), (1,)), ((), ()))
    wqb = wq_ref[...].astype(jnp.bfloat16).astype(jnp.float32)
    wvb = wv_ref[...].astype(jnp.bfloat16).astype(jnp.float32)
    eq_ref[...] = lax.dot_general(m, wqb, dn,
                                  preferred_element_type=jnp.float32,
                                  precision=lax.Precision.HIGHEST)[None]
    ev_ref[...] = lax.dot_general(m, wvb, dn,
                                  preferred_element_type=jnp.float32,
                                  precision=lax.Precision.HIGHEST)[None]


def _embed(x, wq, wv, block):
    n, c, hw = x.shape
    g = n // block
    eq, ev = pl.pallas_call(
        _embed_body,
        grid=(g,),
        in_specs=[
            pl.BlockSpec((block, c, hw), lambda i: (i, 0, 0)),
            pl.BlockSpec((c, c), lambda i: (0, 0)),
            pl.BlockSpec((c, c), lambda i: (0, 0)),
        ],
        out_specs=[
            pl.BlockSpec((1, block, c), lambda i: (i, 0, 0)),
            pl.BlockSpec((1, block, c), lambda i: (i, 0, 0)),
        ],
        out_shape=[
            jax.ShapeDtypeStruct((g, block, c), jnp.float32),
            jax.ShapeDtypeStruct((g, block, c), jnp.float32),
        ],
    )(x, wq, wv)
    return eq.reshape(n, c), ev.reshape(n, c)


def _affinity_body(eqs_ref, eqq_ref, e_ref):
    eq = jnp.concatenate([eqs_ref[...], eqq_ref[...]], axis=0)
    n = eq.shape[0]
    g = lax.dot_general(eq, eq, (((1,), (1,)), ((), ())),
                        preferred_element_type=jnp.float32,
                                  precision=lax.Precision.HIGHEST)
    r = jnp.sum(eq * eq, axis=1)
    w = r[:, None] + r[None, :] - 2.0 * g
    ii = lax.broadcasted_iota(jnp.int32, (n, n), 0)
    jj = lax.broadcasted_iota(jnp.int32, (n, n), 1)
    w = jnp.where(ii == jj, 0.0, jnp.maximum(w, 0.0))
    nz = w != 0.0
    cnt = jnp.sum(nz.astype(jnp.float32))
    mean = jnp.sum(w) / cnt
    var = jnp.sum(jnp.where(nz, (w - mean) ** 2, 0.0)) / (cnt - 1.0)
    std = jnp.sqrt(var)
    e = jnp.exp(-(w / std) / 2.0)
    pad = e_ref.shape[1] - n
    e_ref[...] = jnp.concatenate([e, jnp.zeros((n, pad), jnp.float32)], axis=1)


def _affinity(eq_s, eq_q, npad):
    n = eq_s.shape[0] + eq_q.shape[0]
    return pl.pallas_call(
        _affinity_body,
        out_shape=jax.ShapeDtypeStruct((n, npad), jnp.float32),
    )(eq_s, eq_q)


def _topk(e):
    n, npad = e.shape
    nc, ns = 2, 16
    nw = nc * ns
    rpw = (n + nw - 1) // nw
    nchunks = npad // _L
    mesh = plsc.VectorSubcoreMesh(core_axis_name="c", subcore_axis_name="s",
                                  num_cores=nc, num_subcores=ns)

    @functools.partial(
        pl.kernel,
        mesh=mesh,
        out_type=jax.ShapeDtypeStruct((nw, rpw, _L), jnp.int32),
        scratch_types=[
            pltpu.VMEM((npad,), jnp.float32),
            pltpu.VMEM((npad,), jnp.float32),
            pltpu.VMEM((rpw, _L), jnp.int32),
            pltpu.SemaphoreType.DMA,
            pltpu.SemaphoreType.DMA,
        ],
        compiler_params=pltpu.CompilerParams(needs_layout_passes=False),
    )
    def k(e_hbm, out_hbm, row0, row1, idxb, sem0, sem1):
        wid = lax.axis_index("s") * nc + lax.axis_index("c")
        base = wid * rpw
        nrow = jnp.minimum(rpw, n - base)
        lane = lax.iota(jnp.int32, _L)

        def merge_row(buf):
            rk = buf[pl.ds(0, _L)]
            rv = lane
            rk, rv = plsc.sort_key_val(rk, rv)
            for c in range(1, nchunks):
                ck = buf[pl.ds(c * _L, _L)]
                cv = lane + c * _L
                ck, cv = plsc.sort_key_val(ck, cv, descending=True)
                take = rk >= ck
                nk = jnp.maximum(rk, ck)
                nv = jnp.where(take, rv, cv)
                rk, rv = plsc.sort_key_val(nk, nv)
            return rv

        @pl.when(nrow > 0)
        def _():
            pltpu.make_async_copy(e_hbm.at[base], row0, sem0).start()

        @pl.when(nrow > 1)
        def _():
            pltpu.make_async_copy(e_hbm.at[base + 1], row1, sem1).start()

        def body(it, carry):
            r0 = it * 2
            r1 = r0 + 1

            @pl.when(r0 < nrow)
            def _():
                pltpu.make_async_copy(e_hbm.at[base + r0], row0, sem0).wait()
                idxb[r0, :] = merge_row(row0)

                @pl.when(r0 + 2 < nrow)
                def _():
                    pltpu.make_async_copy(
                        e_hbm.at[base + r0 + 2], row0, sem0).start()

            @pl.when(r1 < nrow)
            def _():
                pltpu.make_async_copy(e_hbm.at[base + r1], row1, sem1).wait()
                idxb[r1, :] = merge_row(row1)

                @pl.when(r1 + 2 < nrow)
                def _():
                    pltpu.make_async_copy(
                        e_hbm.at[base + r1 + 2], row1, sem1).start()

            return carry

        lax.fori_loop(0, (rpw + 1) // 2, body, 0)
        pltpu.sync_copy(idxb, out_hbm.at[wid])

    return k(e).reshape(nw * rpw, _L)[:n]


def _propagate_body(e_ref, idx_ref, evs_ref, evq_ref, out_ref):
    n = idx_ref.shape[0]
    e = e_ref[...][:, :n]
    ev = jnp.concatenate([evs_ref[...], evq_ref[...]], axis=0)
    idx = idx_ref[...]
    jj = lax.broadcasted_iota(jnp.int32, (n, n), 1)
    m = jnp.zeros((n, n), jnp.bool_)
    for k in range(_L - _TOPK, _L):
        m = m | (idx[:, k:k + 1] == jj)
    wm0 = jnp.where(m, e, 0.0)
    wm = jnp.maximum(wm0, wm0.T)
    d = jnp.sum(wm, axis=0)
    dsi = jnp.sqrt(1.0 / (d + _EPS))
    s = dsi[:, None] * wm * dsi[None, :]
    out_ref[...] = lax.dot_general(s, ev, (((1,), (0,)), ((), ())),
                                   preferred_element_type=jnp.float32,
                                  precision=lax.Precision.HIGHEST)


def _propagate(e, idx, ev_s, ev_q):
    n = ev_s.shape[0] + ev_q.shape[0]
    c = ev_s.shape[1]
    return pl.pallas_call(
        _propagate_body,
        out_shape=jax.ShapeDtypeStruct((n, c), jnp.float32),
    )(e, idx, ev_s, ev_q)


def kernel(support, s_labels, query, q_labels, Wq, Wv):
    ns, c = support.shape[0], support.shape[1]
    nq = query.shape[0]
    hw = support.shape[2] * support.shape[3]
    n = ns + nq
    npad = ((n + _L - 1) // _L) * _L
    if npad == n:
        npad += _L
    eq_s, ev_s = _embed(support.reshape(ns, c, hw), Wq, Wv, ns)
    eq_q, ev_q = _embed(query.reshape(nq, c, hw), Wq, Wv, nq // 4)
    e = _affinity(eq_s, eq_q, npad)
    idx = _topk(e)
    return _propagate(e, idx, ev_s, ev_q)
